# Initial kernel scaffold; baseline (speedup 1.0000x reference)
import jax
import jax.numpy as jnp
from jax.experimental import pallas as pl


def kernel(h, r, pos_t, neg_t, entity_table, relation_table, W):
    raise NotImplementedError("write your pallas kernel here")


# speedup vs baseline: 3.7983x; 3.7983x over previous
import functools

import jax
import jax.numpy as jnp
from jax import lax
from jax.experimental import pallas as pl
from jax.experimental.pallas import tpu as pltpu
from jax.experimental.pallas import tpu_sc as plsc

REG_LAMBDA = 0.01
LANES = 128


SUB = 4


def _gather_body(ent_chunks, rel_chunks,
                 eidx_hbm, ridx_hbm, etab_hbm, rtab_hbm,
                 ent_out, rel_out, idx_v, rows_v, sem):
    info = plsc.get_sparse_core_info()
    nc = info.num_cores
    wid = lax.axis_index("s") * nc + lax.axis_index("c")

    def run(idx_hbm, tab_hbm, out_hbm, chunks):
        pltpu.sync_copy(idx_hbm.at[wid], idx_v.at[pl.ds(0, chunks)])
        rows_per_w = chunks * LANES
        for s in range(0, chunks, SUB):
            k = min(SUB, chunks - s)
            copies = []
            for j in range(k):
                copies.append(pltpu.async_copy(
                    tab_hbm.at[idx_v.at[s + j]],
                    rows_v.at[pl.ds(j * LANES, LANES)], sem))
            for c in copies:
                c.wait()
            pltpu.sync_copy(
                rows_v.at[pl.ds(0, k * LANES)],
                out_hbm.at[pl.ds(wid * rows_per_w + s * LANES, k * LANES)])

    run(eidx_hbm, etab_hbm, ent_out, ent_chunks)
    run(ridx_hbm, rtab_hbm, rel_out, rel_chunks)


def _sc_gather(ent_idx, rel_idx, entity_table, relation_table):
    info = plsc.get_sparse_core_info()
    nw = info.num_cores * info.num_subcores
    n_ent = ent_idx.shape[0]
    n_rel = rel_idx.shape[0]
    ent_chunks = n_ent // (nw * LANES)
    rel_chunks = n_rel // (nw * LANES)
    d = entity_table.shape[1]

    eidx = ent_idx.reshape(nw, ent_chunks, LANES)
    ridx = rel_idx.reshape(nw, rel_chunks, LANES)

    mesh = plsc.VectorSubcoreMesh(core_axis_name="c", subcore_axis_name="s")
    kern = functools.partial(
        pl.kernel,
        mesh=mesh,
        out_type=[
            jax.ShapeDtypeStruct((n_ent, d), jnp.float32),
            jax.ShapeDtypeStruct((n_rel, d), jnp.float32),
        ],
        scratch_types=[
            pltpu.VMEM((ent_chunks, LANES), jnp.int32),
            pltpu.VMEM((SUB * LANES, d), jnp.float32),
            pltpu.SemaphoreType.DMA,
        ],
    )(functools.partial(_gather_body, ent_chunks, rel_chunks))
    return kern(eidx, ridx, entity_table, relation_table)


def _loss_body(nb, bsz, gh, gp, gn, gr, w_ref, out_ref, acc_ref):
    i = pl.program_id(0)

    @pl.when(i == 0)
    def _():
        acc_ref[0] = 0.0
        acc_ref[1] = 0.0

    w = w_ref[...]
    he = jnp.abs(jnp.dot(gh[...], w, preferred_element_type=jnp.float32))
    pe = jnp.abs(jnp.dot(gp[...], w, preferred_element_type=jnp.float32))
    ne = jnp.abs(jnp.dot(gn[...], w, preferred_element_type=jnp.float32))
    re = jnp.abs(gr[...])

    base = he + re
    dpos = base - pe
    dneg = base - ne
    pos_s = 0.5 * jnp.sum(dpos * dpos, axis=1, keepdims=True)
    neg_s = 0.5 * jnp.sum(dneg * dneg, axis=1, keepdims=True)
    x = neg_s - pos_s
    logsig = jnp.minimum(x, 0.0) - jnp.log1p(jnp.exp(-jnp.abs(x)))
    sq = (jnp.sum(he * he) + jnp.sum(re * re)
          + jnp.sum(pe * pe) + jnp.sum(ne * ne))
    acc_ref[0] += jnp.sum(logsig)
    acc_ref[1] += sq

    @pl.when(i == nb - 1)
    def _():
        b_total = jnp.float32(nb * bsz)
        out_ref[0, 0] = (-acc_ref[0] / b_total
                         + REG_LAMBDA * acc_ref[1] / (2.0 * b_total))


def _tc_loss(gh, gp, gn, gr, W):
    b, d = gh.shape
    bsz = 2048
    nb = b // bsz
    spec = pl.BlockSpec((bsz, d), lambda i: (i, 0))
    out = pl.pallas_call(
        functools.partial(_loss_body, nb, bsz),
        grid=(nb,),
        in_specs=[spec, spec, spec, spec,
                  pl.BlockSpec((d, d), lambda i: (0, 0))],
        out_specs=pl.BlockSpec(memory_space=pltpu.SMEM),
        out_shape=jax.ShapeDtypeStruct((1, 1), jnp.float32),
        scratch_shapes=[pltpu.SMEM((2,), jnp.float32)],
    )(gh, gp, gn, gr, W)
    return out[0, 0]


def kernel(h, r, pos_t, neg_t, entity_table, relation_table, W):
    b = h.shape[0]
    ent_idx = jnp.concatenate(
        [h.reshape(b), pos_t.reshape(b), neg_t.reshape(b)]).astype(jnp.int32)
    rel_idx = r.reshape(b).astype(jnp.int32)
    ent_rows, rel_rows = _sc_gather(ent_idx, rel_idx,
                                    entity_table, relation_table)
    gh = ent_rows[:b]
    gp = ent_rows[b:2 * b]
    gn = ent_rows[2 * b:]
    return _tc_loss(gh, gp, gn, rel_rows, W)


# speedup vs baseline: 4.6279x; 1.2184x over previous
import functools

import jax
import jax.numpy as jnp
from jax import lax
from jax.experimental import pallas as pl
from jax.experimental.pallas import tpu as pltpu
from jax.experimental.pallas import tpu_sc as plsc

REG_LAMBDA = 0.01
LANES = 128


SUB = 4


def _gather_body(ent_chunks, rel_chunks,
                 eidx_hbm, ridx_hbm, etab_hbm, rtab_hbm,
                 ent_out, rel_out, idx_v, rows_v, sem):
    info = plsc.get_sparse_core_info()
    nc = info.num_cores
    wid = lax.axis_index("s") * nc + lax.axis_index("c")

    def run(idx_hbm, tab_hbm, out_hbm, chunks):
        pltpu.sync_copy(idx_hbm.at[wid], idx_v.at[pl.ds(0, chunks)])
        rows_per_w = chunks * LANES
        for s in range(0, chunks, SUB):
            k = min(SUB, chunks - s)
            copies = []
            for j in range(k):
                copies.append(pltpu.async_copy(
                    tab_hbm.at[idx_v.at[s + j]],
                    rows_v.at[pl.ds(j * LANES, LANES)], sem))
            for c in copies:
                c.wait()
            pltpu.sync_copy(
                rows_v.at[pl.ds(0, k * LANES)],
                out_hbm.at[pl.ds(wid * rows_per_w + s * LANES, k * LANES)])

    run(eidx_hbm, etab_hbm, ent_out, ent_chunks)
    run(ridx_hbm, rtab_hbm, rel_out, rel_chunks)


def _sc_gather(ent_idx, rel_idx, entity_table, relation_table):
    info = plsc.get_sparse_core_info()
    nw = info.num_cores * info.num_subcores
    n_ent = ent_idx.shape[0]
    n_rel = rel_idx.shape[0]
    ent_chunks = n_ent // (nw * LANES)
    rel_chunks = n_rel // (nw * LANES)
    d = entity_table.shape[1]

    eidx = ent_idx.reshape(nw, ent_chunks, LANES)
    ridx = rel_idx.reshape(nw, rel_chunks, LANES)

    mesh = plsc.VectorSubcoreMesh(core_axis_name="c", subcore_axis_name="s")
    kern = functools.partial(
        pl.kernel,
        mesh=mesh,
        out_type=[
            jax.ShapeDtypeStruct((n_ent, d), jnp.float32),
            jax.ShapeDtypeStruct((n_rel, d), jnp.float32),
        ],
        scratch_types=[
            pltpu.VMEM((ent_chunks, LANES), jnp.int32),
            pltpu.VMEM((SUB * LANES, d), jnp.float32),
            pltpu.SemaphoreType.DMA,
        ],
    )(functools.partial(_gather_body, ent_chunks, rel_chunks))
    return kern(eidx, ridx, entity_table, relation_table)


def _loss_body(nb, bsz, gh, gp, gn, gr, w_ref, out_ref, acc_ref):
    i = pl.program_id(0)

    @pl.when(i == 0)
    def _():
        acc_ref[0] = 0.0
        acc_ref[1] = 0.0

    w = w_ref[...]
    he = jnp.abs(jnp.dot(gh[...], w, preferred_element_type=jnp.float32))
    pe = jnp.abs(jnp.dot(gp[...], w, preferred_element_type=jnp.float32))
    ne = jnp.abs(jnp.dot(gn[...], w, preferred_element_type=jnp.float32))
    re = jnp.abs(gr[...])

    base = he + re
    dpos = base - pe
    dneg = base - ne
    pos_s = 0.5 * jnp.sum(dpos * dpos, axis=1, keepdims=True)
    neg_s = 0.5 * jnp.sum(dneg * dneg, axis=1, keepdims=True)
    x = neg_s - pos_s
    logsig = jnp.minimum(x, 0.0) - jnp.log1p(jnp.exp(-jnp.abs(x)))
    sq = (jnp.sum(he * he) + jnp.sum(re * re)
          + jnp.sum(pe * pe) + jnp.sum(ne * ne))
    acc_ref[0] += jnp.sum(logsig)
    acc_ref[1] += sq

    @pl.when(i == nb - 1)
    def _():
        b_total = jnp.float32(nb * bsz)
        out_ref[0, 0] = (-acc_ref[0] / b_total
                         + REG_LAMBDA * acc_ref[1] / (2.0 * b_total))


def _tc_loss(ent_rows, gr, W):
    b, d = gr.shape
    bsz = 2048
    nb = b // bsz
    out = pl.pallas_call(
        functools.partial(_loss_body, nb, bsz),
        grid=(nb,),
        in_specs=[pl.BlockSpec((bsz, d), lambda i: (i, 0)),
                  pl.BlockSpec((bsz, d), lambda i: (nb + i, 0)),
                  pl.BlockSpec((bsz, d), lambda i: (2 * nb + i, 0)),
                  pl.BlockSpec((bsz, d), lambda i: (i, 0)),
                  pl.BlockSpec((d, d), lambda i: (0, 0))],
        out_specs=pl.BlockSpec(memory_space=pltpu.SMEM),
        out_shape=jax.ShapeDtypeStruct((1, 1), jnp.float32),
        scratch_shapes=[pltpu.SMEM((2,), jnp.float32)],
    )(ent_rows, ent_rows, ent_rows, gr, W)
    return out[0, 0]


def kernel(h, r, pos_t, neg_t, entity_table, relation_table, W):
    b = h.shape[0]
    ent_idx = jnp.concatenate(
        [h.reshape(b), pos_t.reshape(b), neg_t.reshape(b)]).astype(jnp.int32)
    rel_idx = r.reshape(b).astype(jnp.int32)
    ent_rows, rel_rows = _sc_gather(ent_idx, rel_idx,
                                    entity_table, relation_table)
    return _tc_loss(ent_rows, rel_rows, W)
